# Initial kernel scaffold; baseline (speedup 1.0000x reference)
import jax
import jax.numpy as jnp
from jax.experimental import pallas as pl


def kernel(input_ids, weight):
    raise NotImplementedError("write your pallas kernel here")


# speedup vs baseline: 1.0567x; 1.0567x over previous
import functools

import jax
import jax.numpy as jnp
from jax import lax
from jax.experimental import pallas as pl
from jax.experimental.pallas import tpu as pltpu
from jax.experimental.pallas import tpu_sc as plsc

VOCAB = 23236
DIM = 1728
BATCH = 1024
SEQ = 50
N = BATCH * SEQ
NC, NS = 2, 16
NW = NC * NS
ROWS_PER_W = N // NW
CH = 32
NCHUNK = ROWS_PER_W // CH


def _emb_body(table_hbm, idx_hbm, out_hbm, idx_v, rows_v, gsem, ssem):
    wid = lax.axis_index("s") * NC + lax.axis_index("c")
    base = wid * ROWS_PER_W

    pltpu.sync_copy(idx_hbm.at[wid], idx_v)

    def gather(c, slot):
        return pltpu.async_copy(table_hbm.at[idx_v.at[c]], rows_v.at[slot], gsem)

    def scatter(c, slot):
        return pltpu.async_copy(
            rows_v.at[slot], out_hbm.at[pl.ds(base + c * CH, CH)], ssem)

    def wait_gather(slot):
        pltpu.make_async_copy(table_hbm.at[idx_v.at[0]], rows_v.at[slot], gsem).wait()

    def wait_scatter(c, slot):
        pltpu.make_async_copy(
            rows_v.at[slot], out_hbm.at[pl.ds(base + c * CH, CH)], ssem).wait()

    gather(0, 0)

    wait_gather(0)
    scatter(0, 0)
    gather(1, 1)

    def pair(t, _):
        c = 2 * t
        wait_gather(0)
        scatter(c, 0)
        wait_scatter(c - 1, 1)
        gather(c + 1, 1)
        wait_gather(1)
        scatter(c + 1, 1)
        wait_scatter(c, 0)
        gather(c + 2, 0)
        return _

    lax.fori_loop(1, NCHUNK // 2 - 1, pair, 0)

    wait_gather(0)
    scatter(NCHUNK - 2, 0)
    wait_scatter(NCHUNK - 3, 1)
    gather(NCHUNK - 1, 1)
    wait_gather(1)
    scatter(NCHUNK - 1, 1)
    wait_scatter(NCHUNK - 2, 0)
    wait_scatter(NCHUNK - 1, 1)


@jax.jit
def _emb(weight, idx):
    mesh = plsc.VectorSubcoreMesh(
        core_axis_name="c", subcore_axis_name="s", num_cores=NC, num_subcores=NS)
    f = pl.kernel(
        _emb_body,
        out_type=jax.ShapeDtypeStruct((N, DIM), jnp.float32),
        mesh=mesh,
        scratch_types=[
            pltpu.VMEM((NCHUNK, CH), jnp.int32),
            pltpu.VMEM((2, CH, DIM), jnp.float32),
            pltpu.SemaphoreType.DMA,
            pltpu.SemaphoreType.DMA,
        ],
        compiler_params=pltpu.CompilerParams(use_tc_tiling_on_sc=False),
    )
    return f(weight, idx)


def kernel(input_ids, weight):
    idx = input_ids.reshape(NW, NCHUNK, CH)
    out = _emb(weight, idx)
    return out.reshape(BATCH, SEQ, DIM)


# speedup vs baseline: 1.0957x; 1.0369x over previous
import functools

import jax
import jax.numpy as jnp
from jax import lax
from jax.experimental import pallas as pl
from jax.experimental.pallas import tpu as pltpu
from jax.experimental.pallas import tpu_sc as plsc

VOCAB = 23236
DIM = 1728
DIM_PAD = 1792
BATCH = 1024
SEQ = 50
N = BATCH * SEQ
NC, NS = 2, 16
NW = NC * NS
ROWS_PER_W = N // NW
CH = 32
NCHUNK = ROWS_PER_W // CH


def _emb_body(table_hbm, idx_hbm, out_hbm, idx_v, rows_v, gsem, ssem):
    wid = lax.axis_index("s") * NC + lax.axis_index("c")
    base = wid * ROWS_PER_W

    pltpu.sync_copy(idx_hbm.at[wid], idx_v)

    def gather(c, slot):
        return pltpu.async_copy(table_hbm.at[idx_v.at[c]], rows_v.at[slot], gsem)

    def scatter(c, slot):
        return pltpu.async_copy(
            rows_v.at[slot], out_hbm.at[pl.ds(base + c * CH, CH)], ssem)

    def wait_gather(slot):
        pltpu.make_async_copy(table_hbm.at[idx_v.at[0]], rows_v.at[slot], gsem).wait()

    def wait_scatter(c, slot):
        pltpu.make_async_copy(
            rows_v.at[slot], out_hbm.at[pl.ds(base + c * CH, CH)], ssem).wait()

    gather(0, 0)

    wait_gather(0)
    scatter(0, 0)
    gather(1, 1)

    def pair(t, _):
        c = 2 * t
        wait_gather(0)
        scatter(c, 0)
        wait_scatter(c - 1, 1)
        gather(c + 1, 1)
        wait_gather(1)
        scatter(c + 1, 1)
        wait_scatter(c, 0)
        gather(c + 2, 0)
        return _

    lax.fori_loop(1, NCHUNK // 2 - 1, pair, 0)

    wait_gather(0)
    scatter(NCHUNK - 2, 0)
    wait_scatter(NCHUNK - 3, 1)
    gather(NCHUNK - 1, 1)
    wait_gather(1)
    scatter(NCHUNK - 1, 1)
    wait_scatter(NCHUNK - 2, 0)
    wait_scatter(NCHUNK - 1, 1)


@jax.jit
def _emb(weight, idx):
    mesh = plsc.VectorSubcoreMesh(
        core_axis_name="c", subcore_axis_name="s", num_cores=NC, num_subcores=NS)
    f = pl.kernel(
        _emb_body,
        out_type=jax.ShapeDtypeStruct((N, DIM_PAD), jnp.float32),
        mesh=mesh,
        scratch_types=[
            pltpu.VMEM((NCHUNK, CH), jnp.int32),
            pltpu.VMEM((2, CH, DIM_PAD), jnp.float32),
            pltpu.SemaphoreType.DMA,
            pltpu.SemaphoreType.DMA,
        ],
    )
    wpad = jnp.pad(weight, ((0, 0), (0, DIM_PAD - DIM)))
    out = f(wpad, idx)
    return out[:, :DIM]


def kernel(input_ids, weight):
    idx = input_ids.reshape(NW, NCHUNK, CH)
    out = _emb(weight, idx)
    return out.reshape(BATCH, SEQ, DIM)


# speedup vs baseline: 1.4922x; 1.3618x over previous
import functools

import jax
import jax.numpy as jnp
from jax import lax
from jax.experimental import pallas as pl
from jax.experimental.pallas import tpu as pltpu
from jax.experimental.pallas import tpu_sc as plsc

VOCAB = 23236
DIM = 1728
DIM_PAD = 1792
BATCH = 1024
SEQ = 50
N = BATCH * SEQ
NC, NS = 2, 16
NW = NC * NS
ROWS_PER_W = N // NW
CH = 32
NCHUNK = ROWS_PER_W // CH

PAD_BR = 256
DEPAD_NB = 4


def _emb_body(table_hbm, idx_hbm, out_hbm, idx_v, rows_v, gsem, ssem):
    wid = lax.axis_index("s") * NC + lax.axis_index("c")
    base = wid * ROWS_PER_W

    pltpu.sync_copy(idx_hbm.at[wid], idx_v)

    def gather(c, slot):
        return pltpu.async_copy(table_hbm.at[idx_v.at[c]], rows_v.at[slot], gsem)

    def scatter(c, slot):
        return pltpu.async_copy(
            rows_v.at[slot], out_hbm.at[pl.ds(base + c * CH, CH)], ssem)

    def wait_gather(slot):
        pltpu.make_async_copy(table_hbm.at[idx_v.at[0]], rows_v.at[slot], gsem).wait()

    def wait_scatter(c, slot):
        pltpu.make_async_copy(
            rows_v.at[slot], out_hbm.at[pl.ds(base + c * CH, CH)], ssem).wait()

    gather(0, 0)

    wait_gather(0)
    scatter(0, 0)
    gather(1, 1)

    def pair(t, _):
        c = 2 * t
        wait_gather(0)
        scatter(c, 0)
        wait_scatter(c - 1, 1)
        gather(c + 1, 1)
        wait_gather(1)
        scatter(c + 1, 1)
        wait_scatter(c, 0)
        gather(c + 2, 0)
        return _

    lax.fori_loop(1, NCHUNK // 2 - 1, pair, 0)

    wait_gather(0)
    scatter(NCHUNK - 2, 0)
    wait_scatter(NCHUNK - 3, 1)
    gather(NCHUNK - 1, 1)
    wait_gather(1)
    scatter(NCHUNK - 1, 1)
    wait_scatter(NCHUNK - 2, 0)
    wait_scatter(NCHUNK - 1, 1)


def _pad_body(w_ref, o_ref):
    o_ref[:, :DIM] = w_ref[...]
    o_ref[:, DIM:] = jnp.zeros((PAD_BR, DIM_PAD - DIM), jnp.float32)


def _depad_body(i_ref, o_ref):
    for i in range(DEPAD_NB):
        o_ref[i] = i_ref[pl.ds(i * SEQ, SEQ), pl.ds(0, DIM)]


@jax.jit
def _emb(weight, idx):
    wpad = pl.pallas_call(
        _pad_body,
        grid=(pl.cdiv(VOCAB, PAD_BR),),
        in_specs=[pl.BlockSpec((PAD_BR, DIM), lambda g: (g, 0))],
        out_specs=pl.BlockSpec((PAD_BR, DIM_PAD), lambda g: (g, 0)),
        out_shape=jax.ShapeDtypeStruct((VOCAB, DIM_PAD), jnp.float32),
    )(weight)

    mesh = plsc.VectorSubcoreMesh(
        core_axis_name="c", subcore_axis_name="s", num_cores=NC, num_subcores=NS)
    f = pl.kernel(
        _emb_body,
        out_type=jax.ShapeDtypeStruct((N, DIM_PAD), jnp.float32),
        mesh=mesh,
        scratch_types=[
            pltpu.VMEM((NCHUNK, CH), jnp.int32),
            pltpu.VMEM((2, CH, DIM_PAD), jnp.float32),
            pltpu.SemaphoreType.DMA,
            pltpu.SemaphoreType.DMA,
        ],
    )
    gathered = f(wpad, idx)

    return pl.pallas_call(
        _depad_body,
        grid=(BATCH // DEPAD_NB,),
        in_specs=[pl.BlockSpec((DEPAD_NB * SEQ, DIM_PAD), lambda g: (g, 0))],
        out_specs=pl.BlockSpec((DEPAD_NB, SEQ, DIM), lambda g: (g, 0, 0)),
        out_shape=jax.ShapeDtypeStruct((BATCH, SEQ, DIM), jnp.float32),
    )(gathered)


def kernel(input_ids, weight):
    idx = input_ids.reshape(NW, NCHUNK, CH)
    return _emb(weight, idx)
